# baseline (speedup 1.0000x reference)
import functools

import jax
import jax.numpy as jnp
from jax import lax
from jax.experimental import pallas as pl
from jax.experimental.pallas import tpu as pltpu
from jax.experimental.pallas import tpu_sc as plsc

N_NODES = 10000
N_EDGES = 320000
D = 128
H = 64

NC = 2
NS = 16
L = 16

EDGES_PER_SUB = N_EDGES // NS
EB = 80
NB = EDGES_PER_SUB // EB
DRAIN_ROWS = 1000
ND = N_NODES // DRAIN_ROWS
ZB_ROWS = 20
UNROLL = 4


def _node_mm_body(h_ref, w_ref, b_ref, ah_ref, dbh_ref, eh_ref):
    acc = jnp.dot(h_ref[...], w_ref[0], preferred_element_type=jnp.float32)
    acc = acc + b_ref[0]
    ah_ref[...] = acc[:, 0 * H:1 * H]
    dbh_ref[...] = acc[:, 1 * H:3 * H]
    eh_ref[...] = acc[:, 3 * H:4 * H]


def _node_matmuls(h, w4, b4):
    bm = 1000
    gm = N_NODES // bm
    outh = jax.ShapeDtypeStruct((NC * N_NODES, H), jnp.float32)
    spec_h = pl.BlockSpec((bm, H), lambda i, c: (c * gm + i, 0))
    return pl.pallas_call(
        _node_mm_body,
        grid=(gm, NC),
        in_specs=[
            pl.BlockSpec((bm, D), lambda i, c: (i, 0)),
            pl.BlockSpec((1, D, 4 * H), lambda i, c: (c, 0, 0)),
            pl.BlockSpec((1, 1, 4 * H), lambda i, c: (c, 0, 0)),
        ],
        out_specs=[
            spec_h,
            pl.BlockSpec((bm, 2 * H), lambda i, c: (c * gm + i, 0)),
            spec_h,
        ],
        out_shape=[
            outh,
            jax.ShapeDtypeStruct((NC * N_NODES, 2 * H), jnp.float32),
            outh,
        ],
    )(h, w4, b4)


def _edge_mm_body(e_ref, w_ref, b_ref, ce_ref):
    ce_ref[...] = (
        jnp.dot(e_ref[...], w_ref[...], preferred_element_type=jnp.float32)
        + b_ref[...]
    )


def _edge_matmul(e, wc, bc):
    bm = 8000
    gm = N_EDGES // bm
    return pl.pallas_call(
        _edge_mm_body,
        grid=(gm,),
        in_specs=[
            pl.BlockSpec((bm, D), lambda i: (i, 0)),
            pl.BlockSpec((D, D), lambda i: (0, 0)),
            pl.BlockSpec((1, D), lambda i: (0, 0)),
        ],
        out_specs=pl.BlockSpec((bm, D), lambda i: (i, 0)),
        out_shape=jax.ShapeDtypeStruct((N_EDGES, D), jnp.float32),
    )(e, wc, bc)


def _sc_body(ei2, dbh_t, eh_t, ce_t,
             eij_out, nd_out,
             il, src_g, dst_g, dst_scat,
             ce_b, dbh_b, eh_b, eij_b,
             zb, acc_sp,
             si, sg_dbh, sg_eh, sg_ce, ss, se):
    c = lax.axis_index("c")
    s = lax.axis_index("s")
    node_off = c * N_NODES
    offs = jnp.full((L,), node_off, jnp.int32)

    r0 = s * DRAIN_ROWS

    @pl.when(s < ND)
    def _init():
        def _zero_row(j, _):
            for k in range(D // L):
                zb[j, pl.ds(k * L, L)] = jnp.zeros((L,), jnp.float32)
            return 0
        lax.fori_loop(0, ZB_ROWS, _zero_row, 0)

        def _zero_chunk(t, _):
            pltpu.sync_copy(zb, acc_sp.at[pl.ds(r0 + t * ZB_ROWS, ZB_ROWS)])
            return 0
        lax.fori_loop(0, DRAIN_ROWS // ZB_ROWS, _zero_chunk, 0)

    plsc.subcore_barrier()

    ebase = s * EDGES_PER_SUB

    def _offset_and_gather(p, e0):
        for i in range(EB // L):
            sl = pl.ds(i * L, L)
            src_g[p][sl] = il[p][0, sl] + offs
            dst_g[p][sl] = il[p][1, sl] + offs
            dst_scat[p][sl] = il[p][1, sl]
        pltpu.async_copy(dbh_t.at[src_g[p]], dbh_b[p], sg_dbh[p])
        pltpu.async_copy(eh_t.at[dst_g[p]], eh_b[p], sg_eh[p])
        pltpu.async_copy(ce_t.at[pl.ds(e0, EB), pl.ds(c * H, H)],
                         ce_b[p], sg_ce[p])

    pltpu.sync_copy(ei2.at[:, pl.ds(ebase, EB)], il[0])
    _offset_and_gather(0, ebase)
    pltpu.async_copy(ei2.at[:, pl.ds(ebase + EB, EB)], il[1], si[1])

    def _slot(p, b):
        q = 1 - p
        e0 = ebase + b * EB
        pltpu.make_async_copy(dbh_t.at[src_g[p]], dbh_b[p], sg_dbh[p]).wait()
        pltpu.make_async_copy(eh_t.at[dst_g[p]], eh_b[p], sg_eh[p]).wait()
        pltpu.make_async_copy(ce_t.at[pl.ds(e0, EB), pl.ds(c * H, H)],
                              ce_b[p], sg_ce[p]).wait()

        @pl.when(b + 1 < NB)
        def _prefetch():
            @pl.when(b >= 1)
            def _ws():
                pltpu.make_async_copy(dbh_b[q], acc_sp.at[dst_scat[q]],
                                      ss[q]).wait()
            pltpu.make_async_copy(ei2.at[:, pl.ds(e0 + EB, EB)],
                                  il[q], si[q]).wait()
            _offset_and_gather(q, e0 + EB)

            @pl.when(b + 2 < NB)
            def _idx2():
                pltpu.async_copy(ei2.at[:, pl.ds(e0 + 2 * EB, EB)],
                                 il[p], si[p])

        @pl.when(b >= 1)
        def _we():
            pltpu.make_async_copy(eij_b,
                                  eij_out.at[pl.ds(e0, EB), pl.ds(c * H, H)],
                                  se).wait()

        def _edge(u, _):
            for d in range(UNROLL):
                j = u * UNROLL + d
                for k in range(H // L):
                    sl = pl.ds(k * L, L)
                    sh = pl.ds(H + k * L, L)
                    x = ce_b[p][j, sl] + dbh_b[p][j, sl] + eh_b[p][j, sl]
                    eij_b[j, sl] = x
                    sg = 1.0 / (1.0 + jnp.exp(-x))
                    dbh_b[p][j, sl] = sg * dbh_b[p][j, sh]
                    dbh_b[p][j, sh] = sg
            return 0

        lax.fori_loop(0, EB // UNROLL, _edge, 0)
        pltpu.async_copy(eij_b, eij_out.at[pl.ds(e0, EB), pl.ds(c * H, H)], se)
        pltpu.async_copy(dbh_b[p], acc_sp.at[dst_scat[p]], ss[p], add=True)
        return 0

    def _pair(g, _):
        _slot(0, 2 * g)
        _slot(1, 2 * g + 1)
        return 0

    lax.fori_loop(0, NB // 2, _pair, 0)
    pltpu.make_async_copy(dbh_b[0], acc_sp.at[dst_scat[0]], ss[0]).wait()
    pltpu.make_async_copy(dbh_b[1], acc_sp.at[dst_scat[1]], ss[1]).wait()
    pltpu.make_async_copy(eij_b, eij_out.at[pl.ds(ebase, EB), pl.ds(c * H, H)],
                          se).wait()
    plsc.subcore_barrier()

    @pl.when(s < ND)
    def _drain():
        pltpu.sync_copy(acc_sp.at[pl.ds(r0, DRAIN_ROWS)],
                        nd_out.at[pl.ds(node_off + r0, DRAIN_ROWS)])


def _sc_message_pass(edge_index, dbh_t, eh_t, ce_t):
    f32 = jnp.float32
    idx2 = [pltpu.VMEM((EB,), jnp.int32)] * 2
    sem2 = [pltpu.SemaphoreType.DMA] * 2
    kern = pl.kernel(
        _sc_body,
        out_type=[
            jax.ShapeDtypeStruct((N_EDGES, D), f32),
            jax.ShapeDtypeStruct((NC * N_NODES, D), f32),
        ],
        mesh=plsc.VectorSubcoreMesh(core_axis_name="c", subcore_axis_name="s"),
        compiler_params=pltpu.CompilerParams(use_tc_tiling_on_sc=False),
        scratch_types=[
            [pltpu.VMEM((2, EB), jnp.int32)] * 2,
            idx2, idx2, idx2,
            [pltpu.VMEM((EB, H), f32)] * 2,
            [pltpu.VMEM((EB, D), f32)] * 2,
            [pltpu.VMEM((EB, H), f32)] * 2,
            pltpu.VMEM((EB, H), f32),
            pltpu.VMEM((ZB_ROWS, D), f32),
            pltpu.VMEM_SHARED((N_NODES, D), f32),
            sem2, sem2, sem2, sem2, sem2,
            pltpu.SemaphoreType.DMA,
        ],
    )
    return kern(edge_index, dbh_t, eh_t, ce_t)


def _h_fin_body(ah_ref, nd_ref, h_ref, g_ref, b_ref, out_ref):
    ah = ah_ref[...]
    nd = nd_ref[...]
    upd_lo = ah[:N_NODES] + nd[:N_NODES, :H] / (nd[:N_NODES, H:] + 1e-6)
    upd_hi = ah[N_NODES:] + nd[N_NODES:, :H] / (nd[N_NODES:, H:] + 1e-6)
    h_upd = jnp.concatenate([upd_lo, upd_hi], axis=1)
    has_in = jnp.max(nd[:N_NODES, H:], axis=1, keepdims=True) > 0.0
    h_new = jnp.where(has_in, h_upd, h_ref[...])
    mu = jnp.mean(h_new, axis=0, keepdims=True)
    xc = h_new - mu
    var = jnp.mean(xc * xc, axis=0, keepdims=True)
    y = g_ref[...] * xc / jnp.sqrt(var + 1e-5) + b_ref[...]
    out_ref[...] = jnp.maximum(y, 0.0)


def _h_finalize(ah, nd, h, gamma_h, beta_h):
    return pl.pallas_call(
        _h_fin_body,
        in_specs=[
            pl.BlockSpec((NC * N_NODES, H), lambda: (0, 0)),
            pl.BlockSpec((NC * N_NODES, D), lambda: (0, 0)),
            pl.BlockSpec((N_NODES, D), lambda: (0, 0)),
            pl.BlockSpec((1, D), lambda: (0, 0)),
            pl.BlockSpec((1, D), lambda: (0, 0)),
        ],
        out_specs=pl.BlockSpec((N_NODES, D), lambda: (0, 0)),
        out_shape=jax.ShapeDtypeStruct((N_NODES, D), jnp.float32),
    )(ah, nd, h, gamma_h.reshape(1, D), beta_h.reshape(1, D))


def _e_fin_body(e_ref, g_ref, b_ref, out_ref, acc_ref):
    t = pl.program_id(0)
    i = pl.program_id(1)

    @pl.when((t == 0) & (i == 0))
    def _zero():
        acc_ref[...] = jnp.zeros_like(acc_ref)

    @pl.when(t == 0)
    def _accum():
        x = e_ref[...]
        acc_ref[0, :] += jnp.sum(x, axis=0)
        acc_ref[1, :] += jnp.sum(x * x, axis=0)

    @pl.when(t == 1)
    def _apply():
        inv_n = 1.0 / N_EDGES
        mu = acc_ref[0, :] * inv_n
        var = acc_ref[1, :] * inv_n - mu * mu
        scale = g_ref[0] / jnp.sqrt(var + 1e-5)
        y = scale[None, :] * (e_ref[...] - mu[None, :]) + b_ref[0][None, :]
        out_ref[...] = jnp.maximum(y, 0.0)


def _e_finalize(eij, gamma_e, beta_e):
    bm = 8000
    gm = N_EDGES // bm
    return pl.pallas_call(
        _e_fin_body,
        grid=(2, gm),
        in_specs=[
            pl.BlockSpec((bm, D), lambda t, i: (i, 0)),
            pl.BlockSpec((1, D), lambda t, i: (0, 0)),
            pl.BlockSpec((1, D), lambda t, i: (0, 0)),
        ],
        out_specs=pl.BlockSpec((bm, D), lambda t, i: (i * t, 0)),
        out_shape=jax.ShapeDtypeStruct((N_EDGES, D), jnp.float32),
        scratch_shapes=[pltpu.VMEM((2, D), jnp.float32)],
    )(eij, gamma_e.reshape(1, D), beta_e.reshape(1, D))


def kernel(h, e, edge_index, W_A, b_A, W_B, b_B, W_C, b_C, W_D, b_D, W_E, b_E,
           gamma_h, beta_h, gamma_e, beta_e):
    w4 = jnp.stack([
        jnp.concatenate([W.T[:, c * H:(c + 1) * H]
                         for W in (W_A, W_D, W_B, W_E)], axis=1)
        for c in range(NC)
    ])
    b4 = jnp.stack([
        jnp.concatenate([b[c * H:(c + 1) * H]
                         for b in (b_A, b_D, b_B, b_E)])
        for c in range(NC)
    ])[:, None, :]
    wc = W_C.T
    bc = b_C.reshape(1, D)

    ah, dbh, eh = _node_matmuls(h, w4, b4)
    ce = _edge_matmul(e, wc, bc)
    eij, nd = _sc_message_pass(edge_index, dbh, eh, ce)
    h_out = _h_finalize(ah, nd, h, gamma_h, beta_h)
    e_out = _e_finalize(eij, gamma_e, beta_e)
    return (h_out, e_out)


# speedup vs baseline: 2.1969x; 2.1969x over previous
import functools

import jax
import jax.numpy as jnp
from jax import lax
from jax.experimental import pallas as pl
from jax.experimental.pallas import tpu as pltpu
from jax.experimental.pallas import tpu_sc as plsc

N_NODES = 10000
N_EDGES = 320000
D = 128
H = 64

NC = 2
NS = 16
L = 16

EDGES_PER_SUB = N_EDGES // NS
EB = 80
NB = EDGES_PER_SUB // EB
DRAIN_ROWS = 1000
ND = N_NODES // DRAIN_ROWS
ZB_ROWS = 40
UNROLL = 4


def _node_mm_body(h_ref, w_ref, b_ref, ah_ref, bh_ref, dh_ref, eh_ref):
    acc = jnp.dot(h_ref[...], w_ref[0], preferred_element_type=jnp.float32)
    acc = acc + b_ref[0]
    ah_ref[...] = acc[:, 0 * H:1 * H]
    bh_ref[...] = acc[:, 1 * H:2 * H]
    dh_ref[...] = acc[:, 2 * H:3 * H]
    eh_ref[...] = acc[:, 3 * H:4 * H]


def _node_matmuls(h, w4, b4):
    bm = 1000
    gm = N_NODES // bm
    out = jax.ShapeDtypeStruct((NC * N_NODES, H), jnp.float32)
    spec_out = pl.BlockSpec((bm, H), lambda i, c: (c * gm + i, 0))
    return pl.pallas_call(
        _node_mm_body,
        grid=(gm, NC),
        in_specs=[
            pl.BlockSpec((bm, D), lambda i, c: (i, 0)),
            pl.BlockSpec((1, D, 4 * H), lambda i, c: (c, 0, 0)),
            pl.BlockSpec((1, 1, 4 * H), lambda i, c: (c, 0, 0)),
        ],
        out_specs=[spec_out] * 4,
        out_shape=[out] * 4,
    )(h, w4, b4)


def _edge_mm_body(e_ref, w_ref, b_ref, ce_ref):
    ce_ref[...] = (
        jnp.dot(e_ref[...], w_ref[...], preferred_element_type=jnp.float32)
        + b_ref[...]
    )


def _edge_matmul(e, wc, bc):
    bm = 2000
    gm = N_EDGES // bm
    return pl.pallas_call(
        _edge_mm_body,
        grid=(gm,),
        in_specs=[
            pl.BlockSpec((bm, D), lambda i: (i, 0)),
            pl.BlockSpec((D, D), lambda i: (0, 0)),
            pl.BlockSpec((1, D), lambda i: (0, 0)),
        ],
        out_specs=pl.BlockSpec((bm, D), lambda i: (i, 0)),
        out_shape=jax.ShapeDtypeStruct((N_EDGES, D), jnp.float32),
    )(e, wc, bc)


def _sc_body(src_h, dst_h, dh_t, eh_t, bh_t, ce_t,
             eij_out, num_out, den_out,
             src_land, dst_land, src_g, dst_g, dst_scat,
             ce_b, dh_b, eh_b, bh_b, eij_b,
             zb, num_sp, den_sp,
             si_src, si_dst, sg_dh, sg_eh, sg_bh, sg_ce,
             ss_num, ss_den, se):
    c = lax.axis_index("c")
    s = lax.axis_index("s")
    node_off = c * N_NODES
    offs = jnp.full((L,), node_off, jnp.int32)

    r0 = s * DRAIN_ROWS

    @pl.when(s < ND)
    def _init():
        def _zero_row(j, _):
            for k in range(H // L):
                zb[j, pl.ds(k * L, L)] = jnp.zeros((L,), jnp.float32)
            return 0
        lax.fori_loop(0, ZB_ROWS, _zero_row, 0)

        def _zero_chunk(t, _):
            rr = r0 + t * ZB_ROWS
            pltpu.sync_copy(zb, num_sp.at[pl.ds(rr, ZB_ROWS)])
            pltpu.sync_copy(zb, den_sp.at[pl.ds(rr, ZB_ROWS)])
            return 0
        lax.fori_loop(0, DRAIN_ROWS // ZB_ROWS, _zero_chunk, 0)

    plsc.subcore_barrier()

    ebase = s * EDGES_PER_SUB

    def _offset_and_gather(p, e0):
        for i in range(EB // L):
            sl = pl.ds(i * L, L)
            src_g[p][sl] = src_land[p][sl] + offs
            dst_g[p][sl] = dst_land[p][sl] + offs
            dst_scat[p][sl] = dst_land[p][sl]
        pltpu.async_copy(dh_t.at[src_g[p]], dh_b[p], sg_dh[p])
        pltpu.async_copy(eh_t.at[dst_g[p]], eh_b[p], sg_eh[p])
        pltpu.async_copy(bh_t.at[src_g[p]], bh_b[p], sg_bh[p])
        pltpu.async_copy(ce_t.at[pl.ds(e0, EB), pl.ds(c * H, H)],
                         ce_b[p], sg_ce[p])

    pltpu.sync_copy(src_h.at[pl.ds(ebase, EB)], src_land[0])
    pltpu.sync_copy(dst_h.at[pl.ds(ebase, EB)], dst_land[0])
    _offset_and_gather(0, ebase)
    pltpu.async_copy(src_h.at[pl.ds(ebase + EB, EB)], src_land[1], si_src[1])
    pltpu.async_copy(dst_h.at[pl.ds(ebase + EB, EB)], dst_land[1], si_dst[1])

    def _slot(p, b):
        q = 1 - p
        e0 = ebase + b * EB
        pltpu.make_async_copy(dh_t.at[src_g[p]], dh_b[p], sg_dh[p]).wait()
        pltpu.make_async_copy(eh_t.at[dst_g[p]], eh_b[p], sg_eh[p]).wait()
        pltpu.make_async_copy(bh_t.at[src_g[p]], bh_b[p], sg_bh[p]).wait()
        pltpu.make_async_copy(ce_t.at[pl.ds(e0, EB), pl.ds(c * H, H)],
                              ce_b[p], sg_ce[p]).wait()

        @pl.when(b + 1 < NB)
        def _prefetch():
            @pl.when(b >= 1)
            def _ws():
                pltpu.make_async_copy(dh_b[q], num_sp.at[dst_scat[q]],
                                      ss_num[q]).wait()
                pltpu.make_async_copy(ce_b[q], den_sp.at[dst_scat[q]],
                                      ss_den[q]).wait()
            pltpu.make_async_copy(src_h.at[pl.ds(e0 + EB, EB)],
                                  src_land[q], si_src[q]).wait()
            pltpu.make_async_copy(dst_h.at[pl.ds(e0 + EB, EB)],
                                  dst_land[q], si_dst[q]).wait()
            _offset_and_gather(q, e0 + EB)

            @pl.when(b + 2 < NB)
            def _idx2():
                pltpu.async_copy(src_h.at[pl.ds(e0 + 2 * EB, EB)],
                                 src_land[p], si_src[p])
                pltpu.async_copy(dst_h.at[pl.ds(e0 + 2 * EB, EB)],
                                 dst_land[p], si_dst[p])

        @pl.when(b >= 1)
        def _we():
            pltpu.make_async_copy(eij_b,
                                  eij_out.at[pl.ds(e0, EB), pl.ds(c * H, H)],
                                  se).wait()

        def _edge(u, _):
            for d in range(UNROLL):
                j = u * UNROLL + d
                for k in range(H // L):
                    sl = pl.ds(k * L, L)
                    x = ce_b[p][j, sl] + dh_b[p][j, sl] + eh_b[p][j, sl]
                    eij_b[j, sl] = x
                    sg = 1.0 / (1.0 + jnp.exp(-x))
                    ce_b[p][j, sl] = sg
                    dh_b[p][j, sl] = sg * bh_b[p][j, sl]
            return 0

        lax.fori_loop(0, EB // UNROLL, _edge, 0)
        pltpu.async_copy(eij_b, eij_out.at[pl.ds(e0, EB), pl.ds(c * H, H)], se)
        pltpu.async_copy(dh_b[p], num_sp.at[dst_scat[p]], ss_num[p], add=True)
        pltpu.async_copy(ce_b[p], den_sp.at[dst_scat[p]], ss_den[p], add=True)
        return 0

    def _pair(g, _):
        _slot(0, 2 * g)
        _slot(1, 2 * g + 1)
        return 0

    lax.fori_loop(0, NB // 2, _pair, 0)
    pltpu.make_async_copy(dh_b[0], num_sp.at[dst_scat[0]], ss_num[0]).wait()
    pltpu.make_async_copy(ce_b[0], den_sp.at[dst_scat[0]], ss_den[0]).wait()
    pltpu.make_async_copy(dh_b[1], num_sp.at[dst_scat[1]], ss_num[1]).wait()
    pltpu.make_async_copy(ce_b[1], den_sp.at[dst_scat[1]], ss_den[1]).wait()
    pltpu.make_async_copy(eij_b, eij_out.at[pl.ds(ebase, EB), pl.ds(c * H, H)],
                          se).wait()
    plsc.subcore_barrier()

    @pl.when(s < ND)
    def _drain():
        pltpu.sync_copy(num_sp.at[pl.ds(r0, DRAIN_ROWS)],
                        num_out.at[pl.ds(node_off + r0, DRAIN_ROWS)])
        pltpu.sync_copy(den_sp.at[pl.ds(r0, DRAIN_ROWS)],
                        den_out.at[pl.ds(node_off + r0, DRAIN_ROWS)])


def _sc_message_pass(edge_index, dh_t, eh_t, bh_t, ce_t):
    f32 = jnp.float32
    idx2 = [pltpu.VMEM((EB,), jnp.int32)] * 2
    buf2 = [pltpu.VMEM((EB, H), f32)] * 2
    sem2 = [pltpu.SemaphoreType.DMA] * 2
    kern = pl.kernel(
        _sc_body,
        out_type=[
            jax.ShapeDtypeStruct((N_EDGES, D), f32),
            jax.ShapeDtypeStruct((NC * N_NODES, H), f32),
            jax.ShapeDtypeStruct((NC * N_NODES, H), f32),
        ],
        mesh=plsc.VectorSubcoreMesh(core_axis_name="c", subcore_axis_name="s"),
        compiler_params=pltpu.CompilerParams(use_tc_tiling_on_sc=False),
        scratch_types=[
            idx2, idx2, idx2, idx2, idx2,
            buf2, buf2, buf2, buf2,
            pltpu.VMEM((EB, H), f32),
            pltpu.VMEM((ZB_ROWS, H), f32),
            pltpu.VMEM_SHARED((N_NODES, H), f32),
            pltpu.VMEM_SHARED((N_NODES, H), f32),
            sem2, sem2, sem2, sem2, sem2, sem2,
            sem2, sem2,
            pltpu.SemaphoreType.DMA,
        ],
    )
    return kern(edge_index[0], edge_index[1], dh_t, eh_t, bh_t, ce_t)


def _h_fin_body(ah_ref, num_ref, den_ref, h_ref, g_ref, b_ref, out_ref):
    ah = ah_ref[...]
    num = num_ref[...]
    den = den_ref[...]
    upd_lo = ah[:N_NODES] + num[:N_NODES] / (den[:N_NODES] + 1e-6)
    upd_hi = ah[N_NODES:] + num[N_NODES:] / (den[N_NODES:] + 1e-6)
    h_upd = jnp.concatenate([upd_lo, upd_hi], axis=1)
    has_in = jnp.max(den[:N_NODES], axis=1, keepdims=True) > 0.0
    h_new = jnp.where(has_in, h_upd, h_ref[...])
    mu = jnp.mean(h_new, axis=0, keepdims=True)
    xc = h_new - mu
    var = jnp.mean(xc * xc, axis=0, keepdims=True)
    y = g_ref[...] * xc / jnp.sqrt(var + 1e-5) + b_ref[...]
    out_ref[...] = jnp.maximum(y, 0.0)


def _h_finalize(ah, num, den, h, gamma_h, beta_h):
    whole2 = pl.BlockSpec((NC * N_NODES, H), lambda: (0, 0))
    return pl.pallas_call(
        _h_fin_body,
        in_specs=[
            whole2, whole2, whole2,
            pl.BlockSpec((N_NODES, D), lambda: (0, 0)),
            pl.BlockSpec((1, D), lambda: (0, 0)),
            pl.BlockSpec((1, D), lambda: (0, 0)),
        ],
        out_specs=pl.BlockSpec((N_NODES, D), lambda: (0, 0)),
        out_shape=jax.ShapeDtypeStruct((N_NODES, D), jnp.float32),
    )(ah, num, den, h, gamma_h.reshape(1, D), beta_h.reshape(1, D))


def _e_fin_body(e_ref, g_ref, b_ref, out_ref, acc_ref):
    t = pl.program_id(0)
    i = pl.program_id(1)

    @pl.when((t == 0) & (i == 0))
    def _zero():
        acc_ref[...] = jnp.zeros_like(acc_ref)

    @pl.when(t == 0)
    def _accum():
        x = e_ref[...]
        acc_ref[0, :] += jnp.sum(x, axis=0)
        acc_ref[1, :] += jnp.sum(x * x, axis=0)

    @pl.when(t == 1)
    def _apply():
        inv_n = 1.0 / N_EDGES
        mu = acc_ref[0, :] * inv_n
        var = acc_ref[1, :] * inv_n - mu * mu
        scale = g_ref[0] / jnp.sqrt(var + 1e-5)
        y = scale[None, :] * (e_ref[...] - mu[None, :]) + b_ref[0][None, :]
        out_ref[...] = jnp.maximum(y, 0.0)


def _e_finalize(eij, gamma_e, beta_e):
    bm = 2000
    gm = N_EDGES // bm
    return pl.pallas_call(
        _e_fin_body,
        grid=(2, gm),
        in_specs=[
            pl.BlockSpec((bm, D), lambda t, i: (i, 0)),
            pl.BlockSpec((1, D), lambda t, i: (0, 0)),
            pl.BlockSpec((1, D), lambda t, i: (0, 0)),
        ],
        out_specs=pl.BlockSpec((bm, D), lambda t, i: (i * t, 0)),
        out_shape=jax.ShapeDtypeStruct((N_EDGES, D), jnp.float32),
        scratch_shapes=[pltpu.VMEM((2, D), jnp.float32)],
    )(eij, gamma_e.reshape(1, D), beta_e.reshape(1, D))


def kernel(h, e, edge_index, W_A, b_A, W_B, b_B, W_C, b_C, W_D, b_D, W_E, b_E,
           gamma_h, beta_h, gamma_e, beta_e):
    w4 = jnp.stack([
        jnp.concatenate([W.T[:, c * H:(c + 1) * H]
                         for W in (W_A, W_B, W_D, W_E)], axis=1)
        for c in range(NC)
    ])
    b4 = jnp.stack([
        jnp.concatenate([b[c * H:(c + 1) * H]
                         for b in (b_A, b_B, b_D, b_E)])
        for c in range(NC)
    ])[:, None, :]
    wc = W_C.T
    bc = b_C.reshape(1, D)

    ah, bh, dh, eh = _node_matmuls(h, w4, b4)
    ce = _edge_matmul(e, wc, bc)
    eij, num, den = _sc_message_pass(edge_index, dh, eh, bh, ce)
    h_out = _h_finalize(ah, num, den, h, gamma_h, beta_h)
    e_out = _e_finalize(eij, gamma_e, beta_e)
    return (h_out, e_out)
